# baseline (speedup 1.0000x reference)
import jax
import jax.numpy as jnp
from jax.experimental import pallas as pl
from jax.experimental.pallas import tpu as pltpu

THRESH = 50.0
KWTA = 16

T = 16
K = 1024
H = 256
L = 256
HBLK = 16
KC = H // HBLK


def _column_kernel(a_ref, w_ref, out_ref, pot_ref):
    c = pl.program_id(0)

    @pl.when(c == 0)
    def _init():
        pot_ref[...] = jnp.zeros_like(pot_ref)

    acc = jnp.zeros((T, K), jnp.float32)
    for hh in range(HBLK):
        acc += jax.lax.dot_general(
            a_ref[:, hh, :], w_ref[:, hh, :],
            dimension_numbers=(((1,), (1,)), ((), ())),
            preferred_element_type=jnp.float32,
        )
    pot_ref[...] += acc

    @pl.when(c == KC - 1)
    def _epilogue():
        raw = pot_ref[...]
        pot = jnp.where(raw > THRESH, raw, 0.0)
        spikes = (pot > 0.0).astype(jnp.float32)
        vmax = jnp.max(pot) * T
        totals = jnp.sum(pot + spikes * vmax, axis=0, keepdims=True)

        iota = jax.lax.broadcasted_iota(jnp.int32, (1, K), 1)
        mask = jnp.zeros((1, K), jnp.float32)
        work = totals
        for _ in range(KWTA):
            m = jnp.max(work)
            idx = jnp.min(jnp.where(work == m, iota, K))
            won = (m > 0.0).astype(jnp.float32)
            sel = (iota == idx)
            mask = mask + jnp.where(sel, won, 0.0)
            work = jnp.where(sel, -jnp.inf, work)

        out_ref[...] = spikes * mask


@jax.jit
def kernel(rec_field, W):
    A = jnp.squeeze(rec_field, 1)
    Wm = jnp.squeeze(W, 1)

    spikes_masked = pl.pallas_call(
        _column_kernel,
        grid=(KC,),
        in_specs=[
            pl.BlockSpec((T, HBLK, L), lambda c: (0, c, 0)),
            pl.BlockSpec((K, HBLK, L), lambda c: (0, c, 0)),
        ],
        out_specs=pl.BlockSpec((T, K), lambda c: (0, 0)),
        out_shape=jax.ShapeDtypeStruct((T, K), jnp.float32),
        scratch_shapes=[pltpu.VMEM((T, K), jnp.float32)],
    )(A, Wm)

    return spikes_masked.reshape(T, K, 1, 1)
